# Initial kernel scaffold; baseline (speedup 1.0000x reference)
import jax
import jax.numpy as jnp
from jax.experimental import pallas as pl


def kernel(x, edge_index, W1, b1, W2, b2, W3, b3, W4, b4):
    raise NotImplementedError("write your pallas kernel here")


# speedup vs baseline: 13.0516x; 13.0516x over previous
import functools

import jax
import jax.numpy as jnp
from jax import lax
from jax.experimental import pallas as pl
from jax.experimental.pallas import tpu as pltpu
from jax.experimental.pallas import tpu_sc as plsc

N_REAL = 10000
NP = 10240
NSUB = 16
CHUNK = 128
FQ = 64
NODE_ROWS_PT = NP // NSUB


def _make_sc_agg(rows_pt: int):
    mesh = plsc.VectorSubcoreMesh(core_axis_name="c", subcore_axis_name="s")

    @functools.partial(
        pl.kernel,
        out_type=jax.ShapeDtypeStruct((2, NP, FQ), jnp.float32),
        mesh=mesh,
        compiler_params=pltpu.CompilerParams(use_tc_tiling_on_sc=False),
        scratch_types=[
            pltpu.VMEM((rows_pt, CHUNK), jnp.int32),
            pltpu.VMEM((rows_pt, CHUNK), jnp.int32),
            pltpu.VMEM((CHUNK, FQ), jnp.float32),
            pltpu.VMEM((CHUNK, FQ), jnp.float32),
            pltpu.VMEM_SHARED((NP, FQ), jnp.float32),
            pltpu.SemaphoreType.DMA,
            pltpu.SemaphoreType.DMA,
        ],
    )
    def agg(table, src_tbl, dst_tbl, zeros, out, src_v, dst_v, gb0, gb1,
            acc, sem0, sem1):
        c = lax.axis_index("c")
        s = lax.axis_index("s")
        gbufs = (gb0, gb1)
        sems = (sem0, sem1)

        nbase = s * NODE_ROWS_PT
        pltpu.sync_copy(zeros.at[pl.ds(nbase, NODE_ROWS_PT)],
                        acc.at[pl.ds(nbase, NODE_ROWS_PT)])
        base = s * rows_pt
        pltpu.sync_copy(src_tbl.at[c, pl.ds(base, rows_pt)], src_v)
        pltpu.sync_copy(dst_tbl.at[c, pl.ds(base, rows_pt)], dst_v)
        for b in range(2):
            pltpu.async_copy(table.at[src_v.at[b]], gbufs[b], sems[b])
        plsc.subcore_barrier()

        def outer(j0, carry):
            for b in range(2):
                j = j0 * 2 + b
                pltpu.make_async_copy(table.at[src_v.at[j]], gbufs[b],
                                      sems[b]).wait()
                pltpu.sync_copy(gbufs[b], acc.at[dst_v.at[j]], add=True)

                @pl.when(j + 2 < rows_pt)
                def _():
                    pltpu.async_copy(table.at[src_v.at[j + 2]], gbufs[b],
                                     sems[b])
            return carry

        lax.fori_loop(0, rows_pt // 2, outer, 0)

        plsc.subcore_barrier()
        pltpu.sync_copy(acc.at[pl.ds(nbase, NODE_ROWS_PT)],
                        out.at[c, pl.ds(nbase, NODE_ROWS_PT)])

    return agg


def _make_sc_deg(rows_pt: int):
    mesh = plsc.VectorSubcoreMesh(core_axis_name="c", subcore_axis_name="s")

    @functools.partial(
        pl.kernel,
        out_type=jax.ShapeDtypeStruct((2, NP, 16), jnp.float32),
        mesh=mesh,
        compiler_params=pltpu.CompilerParams(use_tc_tiling_on_sc=False),
        scratch_types=[
            pltpu.VMEM((rows_pt, CHUNK), jnp.int32),
            pltpu.VMEM((CHUNK, 16), jnp.float32),
            pltpu.VMEM_SHARED((NP, 16), jnp.float32),
        ],
    )
    def deg(ones_hbm, dst_tbl, zeros, out, dst_v, ones_v, acc):
        c = lax.axis_index("c")
        s = lax.axis_index("s")
        nbase = s * NODE_ROWS_PT
        pltpu.sync_copy(zeros.at[pl.ds(nbase, NODE_ROWS_PT)],
                        acc.at[pl.ds(nbase, NODE_ROWS_PT)])
        pltpu.sync_copy(dst_tbl.at[c, pl.ds(s * rows_pt, rows_pt)], dst_v)
        pltpu.sync_copy(ones_hbm, ones_v)
        plsc.subcore_barrier()

        def body(j, carry):
            pltpu.sync_copy(ones_v, acc.at[dst_v.at[j]], add=True)
            return carry

        lax.fori_loop(0, rows_pt, body, 0)
        plsc.subcore_barrier()
        pltpu.sync_copy(acc.at[pl.ds(nbase, NODE_ROWS_PT)],
                        out.at[c, pl.ds(nbase, NODE_ROWS_PT)])

    return deg


_R = 512


def _tc_dis(degp):
    def body(p_ref, o_ref):
        deg = jnp.sum(p_ref[0] + p_ref[1], axis=1, keepdims=True) * (1.0 / 16.0)
        o_ref[...] = lax.rsqrt(deg + 1.0)

    return pl.pallas_call(
        body,
        grid=(NP // _R,),
        in_specs=[pl.BlockSpec((2, _R, 16), lambda r: (0, r, 0))],
        out_specs=pl.BlockSpec((_R, 1), lambda r: (r, 0)),
        out_shape=jax.ShapeDtypeStruct((NP, 1), jnp.float32),
    )(degp)


def _tc_layer1(xp, w1, dis):
    def body(x_ref, w_ref, d_ref, o_ref):
        h = jnp.dot(x_ref[...], w_ref[0],
                    preferred_element_type=jnp.float32)
        o_ref[...] = (h * d_ref[...])[None]

    return pl.pallas_call(
        body,
        grid=(NP // _R, 4),
        in_specs=[
            pl.BlockSpec((_R, 128), lambda r, q: (r, 0)),
            pl.BlockSpec((1, 128, FQ), lambda r, q: (q, 0, 0)),
            pl.BlockSpec((_R, 1), lambda r, q: (r, 0)),
        ],
        out_specs=pl.BlockSpec((1, _R, FQ), lambda r, q: (q, r, 0)),
        out_shape=jax.ShapeDtypeStruct((4, NP, FQ), jnp.float32),
    )(xp, w1, dis)


def _tc_fuse(g01, g23, hp, dis, brow, w):
    def body(a01_ref, a23_ref, h_ref, d_ref, b_ref, w_ref, o_ref):
        d = d_ref[...]
        aggs = (a01_ref, a23_ref)
        zq = [jnp.maximum(d * (aggs[q // 2][q % 2] + h_ref[q]) + b_ref[q],
                          0.0)
              for q in range(4)]
        z = jnp.concatenate(zq, axis=1)
        h = jnp.dot(z, w_ref[0], preferred_element_type=jnp.float32)
        o_ref[...] = (h * d)[None]

    return pl.pallas_call(
        body,
        grid=(NP // _R, 4),
        in_specs=[
            pl.BlockSpec((2, _R, FQ), lambda r, q: (0, r, 0)),
            pl.BlockSpec((2, _R, FQ), lambda r, q: (0, r, 0)),
            pl.BlockSpec((4, _R, FQ), lambda r, q: (0, r, 0)),
            pl.BlockSpec((_R, 1), lambda r, q: (r, 0)),
            pl.BlockSpec((4, 1, FQ), lambda r, q: (0, 0, 0)),
            pl.BlockSpec((1, 256, FQ), lambda r, q: (q, 0, 0)),
        ],
        out_specs=pl.BlockSpec((1, _R, FQ), lambda r, q: (q, r, 0)),
        out_shape=jax.ShapeDtypeStruct((4, NP, FQ), jnp.float32),
    )(g01, g23, hp, dis, brow, w)


def _tc_final(g4, hp, dis, brow):
    def body(a_ref, h_ref, d_ref, b_ref, o_ref):
        s0 = a_ref[0] + h_ref[0]
        s1 = a_ref[1] + h_ref[1]
        o_ref[...] = (d_ref[...] * jnp.concatenate([s0, s1], axis=1)
                      + b_ref[...])

    return pl.pallas_call(
        body,
        grid=(NP // _R,),
        in_specs=[
            pl.BlockSpec((2, _R, FQ), lambda r: (0, r, 0)),
            pl.BlockSpec((2, _R, FQ), lambda r: (0, r, 0)),
            pl.BlockSpec((_R, 1), lambda r: (r, 0)),
            pl.BlockSpec((1, 128), lambda r: (0, 0)),
        ],
        out_specs=pl.BlockSpec((_R, 128), lambda r: (r, 0)),
        out_shape=jax.ShapeDtypeStruct((NP, 128), jnp.float32),
    )(g4, hp, dis, brow)


def _rows_pt(a: int, b: int) -> int:
    r = -(-a // b)
    return -(-r // 8) * 8


def kernel(x, edge_index, W1, b1, W2, b2, W3, b3, W4, b4):
    src = edge_index[0].astype(jnp.int32)
    dst = edge_index[1].astype(jnp.int32)
    e = src.shape[0]

    def pad_edges(total):
        npad = total - e
        ps = jnp.arange(npad, dtype=jnp.int32) % N_REAL
        pd = N_REAL + jnp.arange(npad, dtype=jnp.int32) % (NP - N_REAL)
        return jnp.concatenate([src, ps]), jnp.concatenate([dst, pd])

    ra = _rows_pt(e, NSUB * CHUNK)
    sa, da = pad_edges(ra * NSUB * CHUNK)
    srcQ01 = jnp.stack([sa, sa + NP]).reshape(2, NSUB * ra, CHUNK)
    srcQ23 = jnp.stack([sa + 2 * NP, sa + 3 * NP]).reshape(2, NSUB * ra, CHUNK)
    dstA = jnp.stack([da, da]).reshape(2, NSUB * ra, CHUNK)

    rb = _rows_pt(e, 2 * NSUB * CHUNK)
    _, db = pad_edges(rb * 2 * NSUB * CHUNK)
    dstB = db.reshape(2, NSUB * rb, CHUNK)

    agg = _make_sc_agg(ra)
    agg_d = _make_sc_deg(rb)

    zq = jnp.zeros((NP, FQ), jnp.float32)
    z16 = jnp.zeros((NP, 16), jnp.float32)
    ones16 = jnp.ones((CHUNK, 16), jnp.float32)

    xp = jnp.pad(x, ((0, NP - N_REAL), (0, 0)))
    b1r = b1.reshape(4, 1, FQ)
    b2r = b2.reshape(4, 1, FQ)
    b3r = b3.reshape(4, 1, FQ)
    b4r = b4.reshape(1, 128)
    def wq(w):
        return w.reshape(w.shape[0], 4, FQ).transpose(1, 0, 2)

    w1q = wq(W1)
    w2q = wq(W2)
    w3q = wq(W3)
    w4q = wq(jnp.pad(W4, ((0, 0), (0, 128))))

    degp = agg_d(ones16, dstB, z16)
    dis = _tc_dis(degp)

    def sc_layer(h):
        t = h.reshape(4 * NP, FQ)
        return agg(t, srcQ01, dstA, zq), agg(t, srcQ23, dstA, zq)

    h1 = _tc_layer1(xp, w1q, dis)
    g1a, g1b = sc_layer(h1)
    h2 = _tc_fuse(g1a, g1b, h1, dis, b1r, w2q)
    g2a, g2b = sc_layer(h2)
    h3 = _tc_fuse(g2a, g2b, h2, dis, b2r, w3q)
    g3a, g3b = sc_layer(h3)
    h4 = _tc_fuse(g3a, g3b, h3, dis, b3r, w4q)
    g4 = agg(h4.reshape(4 * NP, FQ), srcQ01, dstA, zq)
    out = _tc_final(g4, h4[:2], dis, b4r)
    return out[:N_REAL]
